# baseline (speedup 1.0000x reference)
import functools

import jax
import jax.numpy as jnp
from jax import lax
from jax.experimental import pallas as pl
from jax.experimental.pallas import tpu as pltpu
from jax.experimental.pallas import tpu_sc as plsc

NNODES = 100000
DIM = 128
BATCH = 16384

_info = plsc.get_sparse_core_info()
_NC, _NS = _info.num_cores, _info.num_subcores
_NW = _NC * _NS
_B_PER_W = BATCH // _NW
_CHUNK = 128
_NCHUNK = _B_PER_W // _CHUNK

_mesh = plsc.VectorSubcoreMesh(core_axis_name="c", subcore_axis_name="s")


@functools.partial(
    pl.kernel,
    mesh=_mesh,
    out_type=(jax.ShapeDtypeStruct((BATCH, DIM), jnp.float32),
              jax.ShapeDtypeStruct((BATCH, DIM), jnp.float32)),
    scratch_types=(
        pltpu.VMEM((_NCHUNK, _CHUNK), jnp.int32),
        pltpu.VMEM((_CHUNK, DIM), jnp.float32),
        pltpu.VMEM((_CHUNK, DIM), jnp.float32),
        pltpu.SemaphoreType.DMA,
        pltpu.SemaphoreType.DMA,
    ),
)
def _lookup(idx_hbm, emb1_hbm, emb2_hbm, out1_hbm, out2_hbm,
            idx_v, r1, r2, s1, s2):
    wid = lax.axis_index("s") * _NC + lax.axis_index("c")
    base = wid * _B_PER_W
    pltpu.sync_copy(idx_hbm.at[wid], idx_v)

    def step(c, carry):
        cp1 = pltpu.async_copy(emb1_hbm.at[idx_v.at[c]], r1, s1)
        cp2 = pltpu.async_copy(emb2_hbm.at[idx_v.at[c]], r2, s2)
        off = base + c * _CHUNK
        cp1.wait()
        pltpu.sync_copy(r1, out1_hbm.at[pl.ds(off, _CHUNK)])
        cp2.wait()
        pltpu.sync_copy(r2, out2_hbm.at[pl.ds(off, _CHUNK)])
        return carry

    lax.fori_loop(0, _NCHUNK, step, 0)


def kernel(idx, emb1, emb2):
    idx_r = idx.astype(jnp.int32).reshape(_NW, _NCHUNK, _CHUNK)
    out1, out2 = _lookup(idx_r, emb1, emb2)
    return (out1, out2)


# speedup vs baseline: 1.0296x; 1.0296x over previous
import functools

import jax
import jax.numpy as jnp
from jax import lax
from jax.experimental import pallas as pl
from jax.experimental.pallas import tpu as pltpu
from jax.experimental.pallas import tpu_sc as plsc

NNODES = 100000
DIM = 128
BATCH = 16384

_info = plsc.get_sparse_core_info()
_NC, _NS = _info.num_cores, _info.num_subcores
_NW = _NC * _NS
_B_PER_W = BATCH // _NW
_CHUNK = 64
_NCHUNK = _B_PER_W // _CHUNK
_NB = 4

_mesh = plsc.VectorSubcoreMesh(core_axis_name="c", subcore_axis_name="s")


@functools.partial(
    pl.kernel,
    mesh=_mesh,
    out_type=(jax.ShapeDtypeStruct((BATCH, DIM), jnp.float32),
              jax.ShapeDtypeStruct((BATCH, DIM), jnp.float32)),
    scratch_types=(
        [pltpu.VMEM((_NCHUNK, _CHUNK), jnp.int32)]
        + [pltpu.VMEM((_CHUNK, DIM), jnp.float32) for _ in range(2 * _NB)]
        + [pltpu.SemaphoreType.DMA for _ in range(4 * _NB)]
    ),
)
def _lookup(idx_hbm, emb1_hbm, emb2_hbm, out1_hbm, out2_hbm, idx_v, *rs):
    bufs1, bufs2 = rs[:_NB], rs[_NB:2 * _NB]
    sems = rs[2 * _NB:]
    g1s, g2s = sems[:_NB], sems[_NB:2 * _NB]
    w1s, w2s = sems[2 * _NB:3 * _NB], sems[3 * _NB:]

    wid = lax.axis_index("s") * _NC + lax.axis_index("c")
    base = wid * _B_PER_W
    pltpu.sync_copy(idx_hbm.at[wid], idx_v)

    gathers = [None] * _NCHUNK
    writes = [None] * _NCHUNK

    def issue_write(c):
        b = c % _NB
        p1, p2 = gathers[c]
        off = base + c * _CHUNK
        p1.wait()
        w1 = pltpu.async_copy(bufs1[b], out1_hbm.at[pl.ds(off, _CHUNK)], w1s[b])
        p2.wait()
        w2 = pltpu.async_copy(bufs2[b], out2_hbm.at[pl.ds(off, _CHUNK)], w2s[b])
        writes[c] = (w1, w2)

    for c in range(_NCHUNK):
        b = c % _NB
        if c >= _NB:
            pw1, pw2 = writes[c - _NB]
            pw1.wait()
            pw2.wait()
        cp1 = pltpu.async_copy(emb1_hbm.at[idx_v.at[c]], bufs1[b], g1s[b])
        cp2 = pltpu.async_copy(emb2_hbm.at[idx_v.at[c]], bufs2[b], g2s[b])
        gathers[c] = (cp1, cp2)
        if c >= 1:
            issue_write(c - 1)

    issue_write(_NCHUNK - 1)
    for c in range(max(0, _NCHUNK - _NB), _NCHUNK):
        pw1, pw2 = writes[c]
        pw1.wait()
        pw2.wait()


def kernel(idx, emb1, emb2):
    idx_r = idx.astype(jnp.int32).reshape(_NW, _NCHUNK, _CHUNK)
    out1, out2 = _lookup(idx_r, emb1, emb2)
    return (out1, out2)


# speedup vs baseline: 1.0421x; 1.0122x over previous
import functools

import jax
import jax.numpy as jnp
from jax import lax
from jax.experimental import pallas as pl
from jax.experimental.pallas import tpu as pltpu
from jax.experimental.pallas import tpu_sc as plsc

NNODES = 100000
DIM = 128
BATCH = 16384

_info = plsc.get_sparse_core_info()
_NC, _NS = _info.num_cores, _info.num_subcores
_NW = _NC * _NS
_B_PER_W = BATCH // _NW
_SIZES = (64, 64, 128, 128, 128)
_STARTS = (0, 64, 128, 256, 384)
_NCHUNK = len(_SIZES)
_NB = 3
_BUFROWS = 128

_mesh = plsc.VectorSubcoreMesh(core_axis_name="c", subcore_axis_name="s")


@functools.partial(
    pl.kernel,
    mesh=_mesh,
    out_type=(jax.ShapeDtypeStruct((BATCH, DIM), jnp.float32),
              jax.ShapeDtypeStruct((BATCH, DIM), jnp.float32)),
    scratch_types=(
        [pltpu.VMEM((_B_PER_W,), jnp.int32)]
        + [pltpu.VMEM((_BUFROWS, DIM), jnp.float32) for _ in range(2 * _NB)]
        + [pltpu.SemaphoreType.DMA for _ in range(4 * _NB)]
    ),
)
def _lookup(idx_hbm, emb1_hbm, emb2_hbm, out1_hbm, out2_hbm, idx_v, *rs):
    bufs1, bufs2 = rs[:_NB], rs[_NB:2 * _NB]
    sems = rs[2 * _NB:]
    g1s, g2s = sems[:_NB], sems[_NB:2 * _NB]
    w1s, w2s = sems[2 * _NB:3 * _NB], sems[3 * _NB:]

    wid = lax.axis_index("s") * _NC + lax.axis_index("c")
    base = wid * _B_PER_W
    pltpu.sync_copy(idx_hbm.at[wid], idx_v)

    gathers = [None] * _NCHUNK
    writes = [None] * _NCHUNK

    def issue_write(c):
        b = c % _NB
        sz = _SIZES[c]
        p1, p2 = gathers[c]
        off = base + _STARTS[c]
        p1.wait()
        w1 = pltpu.async_copy(bufs1[b].at[pl.ds(0, sz)],
                              out1_hbm.at[pl.ds(off, sz)], w1s[b])
        p2.wait()
        w2 = pltpu.async_copy(bufs2[b].at[pl.ds(0, sz)],
                              out2_hbm.at[pl.ds(off, sz)], w2s[b])
        writes[c] = (w1, w2)

    for c in range(_NCHUNK):
        b = c % _NB
        sz = _SIZES[c]
        if c >= _NB:
            pw1, pw2 = writes[c - _NB]
            pw1.wait()
            pw2.wait()
        idx_c = idx_v.at[pl.ds(_STARTS[c], sz)]
        cp1 = pltpu.async_copy(emb1_hbm.at[idx_c], bufs1[b].at[pl.ds(0, sz)],
                               g1s[b])
        cp2 = pltpu.async_copy(emb2_hbm.at[idx_c], bufs2[b].at[pl.ds(0, sz)],
                               g2s[b])
        gathers[c] = (cp1, cp2)
        if c >= 1:
            issue_write(c - 1)

    issue_write(_NCHUNK - 1)
    for c in range(max(0, _NCHUNK - _NB), _NCHUNK):
        pw1, pw2 = writes[c]
        pw1.wait()
        pw2.wait()


def kernel(idx, emb1, emb2):
    idx_r = idx.astype(jnp.int32).reshape(_NW, _B_PER_W)
    out1, out2 = _lookup(idx_r, emb1, emb2)
    return (out1, out2)


# speedup vs baseline: 1.0539x; 1.0114x over previous
import functools

import jax
import jax.numpy as jnp
from jax import lax
from jax.experimental import pallas as pl
from jax.experimental.pallas import tpu as pltpu
from jax.experimental.pallas import tpu_sc as plsc

NNODES = 100000
DIM = 128
BATCH = 16384

_info = plsc.get_sparse_core_info()
_NC, _NS = _info.num_cores, _info.num_subcores
_NW = _NC * _NS
_B_PER_W = BATCH // _NW
_CHUNK = 128
_NCHUNK = _B_PER_W // _CHUNK
_NB = 3

_mesh = plsc.VectorSubcoreMesh(core_axis_name="c", subcore_axis_name="s")


@functools.partial(
    pl.kernel,
    mesh=_mesh,
    out_type=(jax.ShapeDtypeStruct((BATCH, DIM), jnp.float32),
              jax.ShapeDtypeStruct((BATCH, DIM), jnp.float32)),
    scratch_types=(
        [pltpu.VMEM((_NCHUNK, _CHUNK), jnp.int32)]
        + [pltpu.VMEM((_CHUNK, DIM), jnp.float32) for _ in range(2 * _NB)]
        + [pltpu.SemaphoreType.DMA for _ in range(4 * _NB)]
    ),
)
def _lookup(idx_hbm, emb1_hbm, emb2_hbm, out1_hbm, out2_hbm, idx_v, *rs):
    bufs1, bufs2 = rs[:_NB], rs[_NB:2 * _NB]
    sems = rs[2 * _NB:]
    g1s, g2s = sems[:_NB], sems[_NB:2 * _NB]
    w1s, w2s = sems[2 * _NB:3 * _NB], sems[3 * _NB:]

    wid = lax.axis_index("s") * _NC + lax.axis_index("c")
    base = wid * _B_PER_W
    pltpu.sync_copy(idx_hbm.at[wid], idx_v)

    gathers = [None] * _NCHUNK
    writes = [None] * _NCHUNK

    def issue_write(c):
        b = c % _NB
        p1, p2 = gathers[c]
        off = base + c * _CHUNK
        p1.wait()
        w1 = pltpu.async_copy(bufs1[b], out1_hbm.at[pl.ds(off, _CHUNK)], w1s[b])
        p2.wait()
        w2 = pltpu.async_copy(bufs2[b], out2_hbm.at[pl.ds(off, _CHUNK)], w2s[b])
        writes[c] = (w1, w2)

    h = _CHUNK // 2
    h1a = pltpu.async_copy(emb1_hbm.at[idx_v.at[0, pl.ds(0, h)]],
                           bufs1[0].at[pl.ds(0, h)], g1s[0])
    h2a = pltpu.async_copy(emb2_hbm.at[idx_v.at[0, pl.ds(0, h)]],
                           bufs2[0].at[pl.ds(0, h)], g2s[0])
    h1b = pltpu.async_copy(emb1_hbm.at[idx_v.at[0, pl.ds(h, h)]],
                           bufs1[0].at[pl.ds(h, h)], g1s[0])
    h2b = pltpu.async_copy(emb2_hbm.at[idx_v.at[0, pl.ds(h, h)]],
                           bufs2[0].at[pl.ds(h, h)], g2s[0])
    h1a.wait()
    wh1a = pltpu.async_copy(bufs1[0].at[pl.ds(0, h)],
                            out1_hbm.at[pl.ds(base, h)], w1s[0])
    h2a.wait()
    wh2a = pltpu.async_copy(bufs2[0].at[pl.ds(0, h)],
                            out2_hbm.at[pl.ds(base, h)], w2s[0])
    gathers[0] = (h1b, h2b)

    for c in range(1, _NCHUNK):
        b = c % _NB
        if c >= _NB:
            if c - _NB == 0:
                wh1a.wait()
                wh2a.wait()
            pw1, pw2 = writes[c - _NB]
            pw1.wait()
            pw2.wait()
        cp1 = pltpu.async_copy(emb1_hbm.at[idx_v.at[c]], bufs1[b], g1s[b])
        cp2 = pltpu.async_copy(emb2_hbm.at[idx_v.at[c]], bufs2[b], g2s[b])
        gathers[c] = (cp1, cp2)
        if c == 1:
            p1, p2 = gathers[0]
            p1.wait()
            w1 = pltpu.async_copy(bufs1[0].at[pl.ds(h, h)],
                                  out1_hbm.at[pl.ds(base + h, h)], w1s[0])
            p2.wait()
            w2 = pltpu.async_copy(bufs2[0].at[pl.ds(h, h)],
                                  out2_hbm.at[pl.ds(base + h, h)], w2s[0])
            writes[0] = (w1, w2)
        else:
            issue_write(c - 1)

    issue_write(_NCHUNK - 1)
    for c in range(max(0, _NCHUNK - _NB), _NCHUNK):
        pw1, pw2 = writes[c]
        pw1.wait()
        pw2.wait()


def kernel(idx, emb1, emb2):
    idx_r = idx.astype(jnp.int32).reshape(_NW, _NCHUNK, _CHUNK)
    out1, out2 = _lookup(idx_r, emb1, emb2)
    return (out1, out2)


# speedup vs baseline: 1.0720x; 1.0171x over previous
import functools

import jax
import jax.numpy as jnp
from jax import lax
from jax.experimental import pallas as pl
from jax.experimental.pallas import tpu as pltpu
from jax.experimental.pallas import tpu_sc as plsc

NNODES = 100000
DIM = 128
BATCH = 16384

_info = plsc.get_sparse_core_info()
_NC, _NS = _info.num_cores, _info.num_subcores
_NW = _NC * _NS
_B_PER_W = BATCH // _NW
_CHUNK = 128
_NCHUNK = _B_PER_W // _CHUNK
_NB = 3

_mesh = plsc.VectorSubcoreMesh(core_axis_name="c", subcore_axis_name="s")


@functools.partial(
    pl.kernel,
    mesh=_mesh,
    out_type=(jax.ShapeDtypeStruct((BATCH, DIM), jnp.float32),
              jax.ShapeDtypeStruct((BATCH, DIM), jnp.float32)),
    scratch_types=(
        [pltpu.VMEM((_NCHUNK, _CHUNK), jnp.int32)]
        + [pltpu.VMEM((_CHUNK, DIM), jnp.float32) for _ in range(2 * _NB)]
        + [pltpu.SemaphoreType.DMA for _ in range(4 * _NB)]
    ),
)
def _lookup(idx_hbm, emb1_hbm, emb2_hbm, out1_hbm, out2_hbm, idx_v, *rs):
    bufs1, bufs2 = rs[:_NB], rs[_NB:2 * _NB]
    sems = rs[2 * _NB:]
    g1s, g2s = sems[:_NB], sems[_NB:2 * _NB]
    w1s, w2s = sems[2 * _NB:3 * _NB], sems[3 * _NB:]

    wid = lax.axis_index("s") * _NC + lax.axis_index("c")
    base = wid * _B_PER_W
    pltpu.sync_copy(idx_hbm.at[wid], idx_v)

    gathers = [None] * _NCHUNK
    writes = [None] * _NCHUNK

    def issue_write(c):
        b = c % _NB
        p1, p2 = gathers[c]
        off = base + c * _CHUNK
        p1.wait()
        w1 = pltpu.async_copy(bufs1[b], out1_hbm.at[pl.ds(off, _CHUNK)], w1s[b])
        p2.wait()
        w2 = pltpu.async_copy(bufs2[b], out2_hbm.at[pl.ds(off, _CHUNK)], w2s[b])
        writes[c] = (w1, w2)

    for c in range(_NCHUNK):
        b = c % _NB
        if c >= _NB:
            pw1, pw2 = writes[c - _NB]
            pw1.wait()
            pw2.wait()
        cp1 = pltpu.async_copy(emb1_hbm.at[idx_v.at[c]], bufs1[b], g1s[b])
        cp2 = pltpu.async_copy(emb2_hbm.at[idx_v.at[c]], bufs2[b], g2s[b])
        gathers[c] = (cp1, cp2)
        if c >= 1:
            issue_write(c - 1)

    issue_write(_NCHUNK - 1)
    for c in range(max(0, _NCHUNK - _NB), _NCHUNK):
        pw1, pw2 = writes[c]
        pw1.wait()
        pw2.wait()


def kernel(idx, emb1, emb2):
    idx_r = idx.astype(jnp.int32).reshape(_NW, _NCHUNK, _CHUNK)
    out1, out2 = _lookup(idx_r, emb1, emb2)
    return (out1, out2)
